# baseline (speedup 1.0000x reference)
import functools

import jax
import jax.numpy as jnp
from jax import lax
from jax.experimental import pallas as pl
from jax.experimental.pallas import tpu as pltpu
from jax.experimental.pallas import tpu_sc as plsc

_LANES = 16
_IDXW = 128
_HR = 8


@functools.lru_cache(maxsize=None)
def _build(V1, D, BATCH, H, NC, NS):
    NW = NC * NS
    A = D // _HR
    HB = H // _HR
    CB = BATCH // _IDXW
    n_units = HB * CB
    u_per_w = n_units // NW
    assert D % _HR == 0 and H % _HR == 0 and BATCH % _IDXW == 0
    assert n_units % NW == 0 and u_per_w % 2 == 0

    mesh = plsc.VectorSubcoreMesh(core_axis_name="c", subcore_axis_name="s")

    @functools.partial(
        pl.kernel,
        mesh=mesh,
        compiler_params=pltpu.CompilerParams(
            use_tc_tiling_on_sc=False, needs_layout_passes=False),
        out_type=jax.ShapeDtypeStruct((H, A, CB, _HR, _IDXW), jnp.float32),
        scratch_types=[
            pltpu.VMEM((2, _HR, _IDXW), jnp.int32),
            pltpu.VMEM((2, _HR, _IDXW, D), jnp.float32),
            pltpu.SemaphoreType.DMA,
            pltpu.SemaphoreType.DMA,
            pltpu.SemaphoreType.DMA,
            pltpu.SemaphoreType.DMA,
            pltpu.SemaphoreType.DMA,
        ],
    )
    def emb(idx_hbm, table_hbm, out_hbm, idx_v, rows_v, gat_sem,
            i_sem0, i_sem1, o_sem0, o_sem1):
        wid = lax.axis_index("s") * NC + lax.axis_index("c")
        u0 = wid * u_per_w
        i_sems = (i_sem0, i_sem1)
        o_sems = (o_sem0, o_sem1)

        def unit_coords(u):
            return u // CB, lax.rem(u, CB)

        for p in range(2):
            hb, bb = unit_coords(u0 + p)
            pltpu.async_copy(idx_hbm.at[hb, bb], idx_v.at[p], i_sems[p])

        def outer(t, carry):
            for p in range(2):
                u = u0 + 2 * t + p
                hb, bb = unit_coords(u)

                @pl.when(t > 0)
                def _wait_stores(p=p):
                    for c in range(D):
                        pltpu.make_async_copy(
                            rows_v.at[p, :, :, c],
                            out_hbm.at[pl.ds(0, _HR), 0, 0, 0],
                            o_sems[p]).wait()

                pltpu.make_async_copy(
                    idx_hbm.at[0, 0], idx_v.at[p], i_sems[p]).wait()

                for j in range(_HR):
                    pltpu.async_copy(
                        table_hbm.at[idx_v.at[p].at[j]],
                        rows_v.at[p, j],
                        gat_sem,
                    )
                for j in range(_HR):
                    pltpu.make_async_copy(
                        table_hbm.at[idx_v.at[p].at[0]],
                        rows_v.at[p, 0],
                        gat_sem).wait()

                n_groups = _HR * (_IDXW // _LANES)

                def accmin(gi, mn):
                    j = gi // (_IDXW // _LANES)
                    o = lax.rem(gi, _IDXW // _LANES)
                    return jnp.minimum(
                        mn, idx_v[p, j, pl.ds(o * _LANES, _LANES)])

                mn = lax.fori_loop(
                    1, n_groups, accmin, idx_v[p, 0, pl.ds(0, _LANES)])
                has_pad = jnp.min(mn) == 0

                @pl.when(has_pad)
                def _fixup(p=p):
                    def group(gi, acc):
                        j = gi // (_IDXW // _LANES)
                        o = lax.rem(gi, _IDXW // _LANES)
                        v = idx_v[p, j, pl.ds(o * _LANES, _LANES)]

                        @pl.when(jnp.min(v) == 0)
                        def _zero(j=j, o=o, v=v):
                            m = v == 0
                            pj = jnp.zeros((_LANES,), jnp.int32) + p
                            jj = jnp.zeros((_LANES,), jnp.int32) + j
                            bmv = lax.iota(jnp.int32, _LANES) + o * _LANES
                            z = jnp.zeros((_LANES,), jnp.float32)

                            def zcol(c, acc2):
                                cols = jnp.zeros((_LANES,), jnp.int32) + c
                                plsc.store_scatter(
                                    rows_v, [pj, jj, bmv, cols], z, mask=m)
                                return acc2

                            lax.fori_loop(0, D, zcol, 0)
                        return acc

                    lax.fori_loop(0, n_groups, group, 0)

                @pl.when(2 * t + p + 2 < u_per_w)
                def _prefetch(p=p, u=u):
                    hb2, bb2 = unit_coords(u + 2)
                    pltpu.async_copy(idx_hbm.at[hb2, bb2], idx_v.at[p],
                                     i_sems[p])

                for a in range(A):
                    for r in range(_HR):
                        c = a * _HR + r
                        pltpu.async_copy(
                            rows_v.at[p, :, :, c],
                            out_hbm.at[pl.ds(hb * _HR, _HR), a, bb, r],
                            o_sems[p])
            return carry

        lax.fori_loop(0, u_per_w // 2, outer, 0)

        for p in range(2):
            for c in range(D):
                pltpu.make_async_copy(
                    rows_v.at[p, :, :, c],
                    out_hbm.at[pl.ds(0, _HR), 0, 0, 0],
                    o_sems[p]).wait()

    return emb


def kernel(feature_ids, table):
    batch, hist = feature_ids.shape
    V1, D = table.shape
    ids32 = feature_ids.astype(jnp.int32)
    i4 = ids32.reshape(batch // _IDXW, _IDXW, hist // _HR, _HR)
    i4 = i4.transpose(2, 0, 3, 1)
    info = plsc.get_sparse_core_info()
    emb = _build(V1, D, batch, hist, info.num_cores, info.num_subcores)
    out5 = emb(i4, table)
    out = out5.transpose(2, 4, 0, 1, 3).reshape(batch, hist, D)
    return out


# speedup vs baseline: 136.1817x; 136.1817x over previous
import functools

import jax
import jax.numpy as jnp
from jax import lax
from jax.experimental import pallas as pl
from jax.experimental.pallas import tpu as pltpu
from jax.experimental.pallas import tpu_sc as plsc

_LANES = 16
_IDXW = 128
_HR = 8
_PAD = 135


@functools.lru_cache(maxsize=None)
def _build(V1, D, BATCH, H, NC, NS):
    NW = NC * NS
    A = D // _HR
    HB = H // _HR
    CB = BATCH // _IDXW
    n_units = HB * CB
    u_per_w = n_units // NW
    assert D % _HR == 0 and H % _HR == 0 and BATCH % _IDXW == 0
    assert n_units % NW == 0 and u_per_w % 2 == 0

    mesh = plsc.VectorSubcoreMesh(core_axis_name="c", subcore_axis_name="s")

    @functools.partial(
        pl.kernel,
        mesh=mesh,
        compiler_params=pltpu.CompilerParams(
            use_tc_tiling_on_sc=False, needs_layout_passes=False),
        out_type=jax.ShapeDtypeStruct((H, A, CB, _HR, _IDXW), jnp.float32),
        scratch_types=[
            pltpu.VMEM((2, _HR, _IDXW), jnp.int32),
            pltpu.VMEM((_HR * _IDXW, D), jnp.float32),
            pltpu.VMEM((2, _HR, A, 1, _HR, _PAD), jnp.float32),
            pltpu.SemaphoreType.DMA,
            pltpu.SemaphoreType.DMA,
            pltpu.SemaphoreType.DMA,
            pltpu.SemaphoreType.DMA,
            pltpu.SemaphoreType.DMA,
        ],
    )
    def emb(idx_hbm, table_hbm, out_hbm, idx_v, rows_v, val_v, gat_sem,
            i_sem0, i_sem1, o_sem0, o_sem1):
        wid = lax.axis_index("s") * NC + lax.axis_index("c")
        u0 = wid * u_per_w
        i_sems = (i_sem0, i_sem1)
        o_sems = (o_sem0, o_sem1)

        def unit_coords(u):
            return u // CB, lax.rem(u, CB)

        for p in range(2):
            hb, bb = unit_coords(u0 + p)
            pltpu.async_copy(idx_hbm.at[hb, bb], idx_v.at[p], i_sems[p])

        def outer(t, carry):
            for p in range(2):
                u = u0 + 2 * t + p
                hb, bb = unit_coords(u)

                pltpu.make_async_copy(
                    idx_hbm.at[0, 0], idx_v.at[p], i_sems[p]).wait()

                for j in range(_HR):
                    pltpu.async_copy(
                        table_hbm.at[idx_v.at[p].at[j]],
                        rows_v.at[pl.ds(j * _IDXW, _IDXW)],
                        gat_sem,
                    )
                for j in range(_HR):
                    pltpu.make_async_copy(
                        table_hbm.at[idx_v.at[p].at[0]],
                        rows_v.at[pl.ds(0, _IDXW)],
                        gat_sem).wait()

                n_groups = _HR * (_IDXW // _LANES)

                def accmin(gi, mn):
                    j = gi // (_IDXW // _LANES)
                    o = lax.rem(gi, _IDXW // _LANES)
                    return jnp.minimum(
                        mn, idx_v[p, j, pl.ds(o * _LANES, _LANES)])

                mn = lax.fori_loop(
                    1, n_groups, accmin, idx_v[p, 0, pl.ds(0, _LANES)])

                @pl.when(jnp.min(mn) == 0)
                def _fixup(p=p):
                    def group(gi, acc):
                        j = gi // (_IDXW // _LANES)
                        o = lax.rem(gi, _IDXW // _LANES)
                        v = idx_v[p, j, pl.ds(o * _LANES, _LANES)]

                        @pl.when(jnp.min(v) == 0)
                        def _zero(j=j, o=o, v=v):
                            m = v == 0
                            rowv = j * _IDXW + o * _LANES + lax.iota(
                                jnp.int32, _LANES)
                            z = jnp.zeros((_LANES,), jnp.float32)

                            def zcol(c, acc2):
                                cols = jnp.zeros((_LANES,), jnp.int32) + c
                                plsc.store_scatter(
                                    rows_v, [rowv, cols], z, mask=m)
                                return acc2

                            lax.fori_loop(0, D, zcol, 0)
                        return acc

                    lax.fori_loop(0, n_groups, group, 0)

                @pl.when(t > 0)
                def _wait_store(p=p):
                    pltpu.make_async_copy(
                        val_v.at[p, :, :, :, :, pl.ds(0, _IDXW)],
                        out_hbm.at[pl.ds(0, _HR), :, pl.ds(0, 1)],
                        o_sems[p]).wait()

                def block(j, carry2):
                    jsplat = jnp.zeros((_LANES,), jnp.int32) + j
                    psplat = jnp.zeros((_LANES,), jnp.int32) + p
                    zsplat = jnp.zeros((_LANES,), jnp.int32)
                    for half in range(D // _LANES):
                        dlane = half * _LANES + lax.iota(jnp.int32, _LANES)
                        a_idx = dlane // _HR
                        r_idx = lax.rem(dlane, _HR)
                        for bm in range(_IDXW):
                            v = rows_v[j * _IDXW + bm,
                                       pl.ds(half * _LANES, _LANES)]
                            bsplat = jnp.zeros((_LANES,), jnp.int32) + bm
                            plsc.store_scatter(
                                val_v,
                                [psplat, jsplat, a_idx, zsplat, r_idx, bsplat],
                                v)
                    return carry2

                lax.fori_loop(0, _HR, block, 0)

                @pl.when(2 * t + p + 2 < u_per_w)
                def _prefetch(p=p, u=u):
                    hb2, bb2 = unit_coords(u + 2)
                    pltpu.async_copy(idx_hbm.at[hb2, bb2], idx_v.at[p],
                                     i_sems[p])

                pltpu.async_copy(
                    val_v.at[p, :, :, :, :, pl.ds(0, _IDXW)],
                    out_hbm.at[pl.ds(hb * _HR, _HR), :, pl.ds(bb, 1)],
                    o_sems[p])
            return carry

        lax.fori_loop(0, u_per_w // 2, outer, 0)

        for p in range(2):
            pltpu.make_async_copy(
                val_v.at[p, :, :, :, :, pl.ds(0, _IDXW)],
                out_hbm.at[pl.ds(0, _HR), :, pl.ds(0, 1)],
                o_sems[p]).wait()

    return emb


def kernel(feature_ids, table):
    batch, hist = feature_ids.shape
    V1, D = table.shape
    ids32 = feature_ids.astype(jnp.int32)
    i4 = ids32.reshape(batch // _IDXW, _IDXW, hist // _HR, _HR)
    i4 = i4.transpose(2, 0, 3, 1)
    info = plsc.get_sparse_core_info()
    emb = _build(V1, D, batch, hist, info.num_cores, info.num_subcores)
    out5 = emb(i4, table)
    out = out5.transpose(2, 4, 0, 1, 3).reshape(batch, hist, D)
    return out


# speedup vs baseline: 145.7153x; 1.0700x over previous
import functools

import jax
import jax.numpy as jnp
from jax import lax
from jax.experimental import pallas as pl
from jax.experimental.pallas import tpu as pltpu
from jax.experimental.pallas import tpu_sc as plsc

_LANES = 16
_IDXW = 128
_HR = 8
_PAD = 135


@functools.lru_cache(maxsize=None)
def _build(V1, D, BATCH, H, NC, NS):
    NW = NC * NS
    A = D // _HR
    HB = H // _HR
    CB = BATCH // _IDXW
    n_units = HB * CB
    u_per_w = n_units // NW
    assert D % _HR == 0 and H % _HR == 0 and BATCH % _IDXW == 0
    assert n_units % NW == 0 and u_per_w % 2 == 0

    mesh = plsc.VectorSubcoreMesh(core_axis_name="c", subcore_axis_name="s")

    @functools.partial(
        pl.kernel,
        mesh=mesh,
        compiler_params=pltpu.CompilerParams(
            use_tc_tiling_on_sc=False, needs_layout_passes=False),
        out_type=jax.ShapeDtypeStruct((H, A, CB, _HR, _IDXW), jnp.float32),
        scratch_types=[
            pltpu.VMEM((2, _HR, _IDXW), jnp.int32),
            pltpu.VMEM((_HR * _IDXW, D), jnp.float32),
            pltpu.VMEM((2, _HR, A, 1, _HR, _PAD), jnp.float32),
            pltpu.SemaphoreType.DMA,
            pltpu.SemaphoreType.DMA,
            pltpu.SemaphoreType.DMA,
            pltpu.SemaphoreType.DMA,
            pltpu.SemaphoreType.DMA,
        ],
    )
    def emb(idx_hbm, table_hbm, out_hbm, idx_v, rows_v, val_v, gat_sem,
            i_sem0, i_sem1, o_sem0, o_sem1):
        wid = lax.axis_index("s") * NC + lax.axis_index("c")
        u0 = wid * u_per_w
        i_sems = (i_sem0, i_sem1)
        o_sems = (o_sem0, o_sem1)

        def unit_coords(u):
            return u // CB, lax.rem(u, CB)

        for p in range(2):
            hb, bb = unit_coords(u0 + p)
            pltpu.async_copy(idx_hbm.at[hb, bb], idx_v.at[p], i_sems[p])

        def outer(t, carry):
            for p in range(2):
                u = u0 + 2 * t + p
                hb, bb = unit_coords(u)

                pltpu.make_async_copy(
                    idx_hbm.at[0, 0], idx_v.at[p], i_sems[p]).wait()

                for j in range(_HR):
                    pltpu.async_copy(
                        table_hbm.at[idx_v.at[p].at[j]],
                        rows_v.at[pl.ds(j * _IDXW, _IDXW)],
                        gat_sem,
                    )
                for j in range(_HR):
                    pltpu.make_async_copy(
                        table_hbm.at[idx_v.at[p].at[0]],
                        rows_v.at[pl.ds(0, _IDXW)],
                        gat_sem).wait()

                n_groups = _HR * (_IDXW // _LANES)

                def accmin(gi, mn):
                    j = gi // (_IDXW // _LANES)
                    o = lax.rem(gi, _IDXW // _LANES)
                    return jnp.minimum(
                        mn, idx_v[p, j, pl.ds(o * _LANES, _LANES)])

                mn = lax.fori_loop(
                    1, n_groups, accmin, idx_v[p, 0, pl.ds(0, _LANES)])

                @pl.when(jnp.min(mn) == 0)
                def _fixup(p=p):
                    def group(gi, acc):
                        j = gi // (_IDXW // _LANES)
                        o = lax.rem(gi, _IDXW // _LANES)
                        v = idx_v[p, j, pl.ds(o * _LANES, _LANES)]

                        @pl.when(jnp.min(v) == 0)
                        def _zero(j=j, o=o, v=v):
                            m = v == 0
                            rowv = j * _IDXW + o * _LANES + lax.iota(
                                jnp.int32, _LANES)
                            z = jnp.zeros((_LANES,), jnp.float32)

                            def zcol(c, acc2):
                                cols = jnp.zeros((_LANES,), jnp.int32) + c
                                plsc.store_scatter(
                                    rows_v, [rowv, cols], z, mask=m)
                                return acc2

                            lax.fori_loop(0, D, zcol, 0)
                        return acc

                    lax.fori_loop(0, n_groups, group, 0)

                @pl.when(t > 0)
                def _wait_store(p=p):
                    pltpu.make_async_copy(
                        val_v.at[p, :, :, :, :, pl.ds(0, _IDXW)],
                        out_hbm.at[pl.ds(0, _HR), :, pl.ds(0, 1)],
                        o_sems[p]).wait()

                def block(j, carry2):
                    zsplat = jnp.zeros((_LANES,), jnp.int32)
                    vref = val_v.at[p, j]
                    for half in range(D // _LANES):
                        dlane = half * _LANES + lax.iota(jnp.int32, _LANES)
                        a_idx = dlane // _HR
                        r_idx = lax.rem(dlane, _HR)
                        for bm in range(_IDXW):
                            v = rows_v[j * _IDXW + bm,
                                       pl.ds(half * _LANES, _LANES)]
                            bsplat = zsplat + bm
                            plsc.store_scatter(
                                vref, [a_idx, zsplat, r_idx, bsplat], v)
                    return carry2

                lax.fori_loop(0, _HR, block, 0)

                @pl.when(2 * t + p + 2 < u_per_w)
                def _prefetch(p=p, u=u):
                    hb2, bb2 = unit_coords(u + 2)
                    pltpu.async_copy(idx_hbm.at[hb2, bb2], idx_v.at[p],
                                     i_sems[p])

                pltpu.async_copy(
                    val_v.at[p, :, :, :, :, pl.ds(0, _IDXW)],
                    out_hbm.at[pl.ds(hb * _HR, _HR), :, pl.ds(bb, 1)],
                    o_sems[p])
            return carry

        lax.fori_loop(0, u_per_w // 2, outer, 0)

        for p in range(2):
            pltpu.make_async_copy(
                val_v.at[p, :, :, :, :, pl.ds(0, _IDXW)],
                out_hbm.at[pl.ds(0, _HR), :, pl.ds(0, 1)],
                o_sems[p]).wait()

    return emb


def kernel(feature_ids, table):
    batch, hist = feature_ids.shape
    V1, D = table.shape
    ids32 = feature_ids.astype(jnp.int32)
    i4 = ids32.reshape(batch // _IDXW, _IDXW, hist // _HR, _HR)
    i4 = i4.transpose(2, 0, 3, 1)
    info = plsc.get_sparse_core_info()
    emb = _build(V1, D, batch, hist, info.num_cores, info.num_subcores)
    out5 = emb(i4, table)
    out = out5.transpose(2, 4, 0, 1, 3).reshape(batch, hist, D)
    return out


# speedup vs baseline: 280.5043x; 1.9250x over previous
import functools

import jax
import jax.numpy as jnp
from jax import lax
from jax.experimental import pallas as pl
from jax.experimental.pallas import tpu as pltpu
from jax.experimental.pallas import tpu_sc as plsc

_LANES = 16
_IDXW = 128
_HR = 8
_PAD = 135


@functools.lru_cache(maxsize=None)
def _build(V1, D, BATCH, H, NC, NS):
    NW = NC * NS
    A = D // _HR
    HB = H // _HR
    CB = BATCH // _IDXW
    n_units = HB * CB
    u_per_w = n_units // NW
    assert D % _HR == 0 and H % _HR == 0 and BATCH % _IDXW == 0
    assert n_units % NW == 0 and u_per_w % 2 == 0

    mesh = plsc.VectorSubcoreMesh(core_axis_name="c", subcore_axis_name="s")

    @functools.partial(
        pl.kernel,
        mesh=mesh,
        compiler_params=pltpu.CompilerParams(
            use_tc_tiling_on_sc=False, needs_layout_passes=False),
        out_type=jax.ShapeDtypeStruct((H, A, CB, _HR, _IDXW), jnp.float32),
        scratch_types=[
            pltpu.VMEM((2, _HR, _IDXW), jnp.int32),
            pltpu.VMEM((_HR * _IDXW, D), jnp.float32),
            pltpu.VMEM((2, _HR, A, 1, _HR, _PAD), jnp.float32),
            pltpu.SemaphoreType.DMA,
            pltpu.SemaphoreType.DMA,
            pltpu.SemaphoreType.DMA,
            pltpu.SemaphoreType.DMA,
            pltpu.SemaphoreType.DMA,
        ],
    )
    def emb(idx_hbm, table_hbm, out_hbm, idx_v, rows_v, val_v, gat_sem,
            i_sem0, i_sem1, o_sem0, o_sem1):
        wid = lax.axis_index("s") * NC + lax.axis_index("c")
        u0 = wid * u_per_w
        i_sems = (i_sem0, i_sem1)
        o_sems = (o_sem0, o_sem1)

        def unit_coords(u):
            return u // CB, lax.rem(u, CB)

        for p in range(2):
            hb, bb = unit_coords(u0 + p)
            pltpu.async_copy(idx_hbm.at[hb, bb], idx_v.at[p], i_sems[p])

        def outer(t, carry):
            for p in range(2):
                u = u0 + 2 * t + p
                hb, bb = unit_coords(u)

                pltpu.make_async_copy(
                    idx_hbm.at[0, 0], idx_v.at[p], i_sems[p]).wait()

                for j in range(_HR):
                    pltpu.async_copy(
                        table_hbm.at[idx_v.at[p].at[j]],
                        rows_v.at[pl.ds(j * _IDXW, _IDXW)],
                        gat_sem,
                    )
                for j in range(_HR):
                    pltpu.make_async_copy(
                        table_hbm.at[idx_v.at[p].at[0]],
                        rows_v.at[pl.ds(0, _IDXW)],
                        gat_sem).wait()

                n_groups = _HR * (_IDXW // _LANES)

                def accmin(gi, mn):
                    j = gi // (_IDXW // _LANES)
                    o = lax.rem(gi, _IDXW // _LANES)
                    return jnp.minimum(
                        mn, idx_v[p, j, pl.ds(o * _LANES, _LANES)])

                mn = lax.fori_loop(
                    1, n_groups, accmin, idx_v[p, 0, pl.ds(0, _LANES)])

                @pl.when(jnp.min(mn) == 0)
                def _fixup(p=p):
                    def group(gi, acc):
                        j = gi // (_IDXW // _LANES)
                        o = lax.rem(gi, _IDXW // _LANES)
                        v = idx_v[p, j, pl.ds(o * _LANES, _LANES)]

                        @pl.when(jnp.min(v) == 0)
                        def _zero(j=j, o=o, v=v):
                            m = v == 0
                            rowv = j * _IDXW + o * _LANES + lax.iota(
                                jnp.int32, _LANES)
                            z = jnp.zeros((_LANES,), jnp.float32)

                            def zcol(c, acc2):
                                cols = jnp.zeros((_LANES,), jnp.int32) + c
                                plsc.store_scatter(
                                    rows_v, [rowv, cols], z, mask=m)
                                return acc2

                            lax.fori_loop(0, D, zcol, 0)
                        return acc

                    lax.fori_loop(0, n_groups, group, 0)

                @pl.when(t > 0)
                def _wait_store(p=p):
                    pltpu.make_async_copy(
                        val_v.at[p, :, :, :, :, pl.ds(0, _IDXW)],
                        out_hbm.at[pl.ds(0, _HR), :, pl.ds(0, 1)],
                        o_sems[p]).wait()

                @functools.partial(plsc.parallel_loop, 0, _HR, unroll=2)
                def block(j):
                    zsplat = jnp.zeros((_LANES,), jnp.int32)
                    vref = val_v.at[p, j]
                    rref = rows_v.at[pl.ds(j * _IDXW, _IDXW)]
                    for half in range(D // _LANES):
                        dlane = half * _LANES + lax.iota(jnp.int32, _LANES)
                        a_idx = dlane // _HR
                        r_idx = lax.rem(dlane, _HR)
                        for bm in range(_IDXW):
                            v = rref[bm, pl.ds(half * _LANES, _LANES)]
                            bsplat = zsplat + bm
                            plsc.store_scatter(
                                vref, [a_idx, zsplat, r_idx, bsplat], v)

                @pl.when(2 * t + p + 2 < u_per_w)
                def _prefetch(p=p, u=u):
                    hb2, bb2 = unit_coords(u + 2)
                    pltpu.async_copy(idx_hbm.at[hb2, bb2], idx_v.at[p],
                                     i_sems[p])

                pltpu.async_copy(
                    val_v.at[p, :, :, :, :, pl.ds(0, _IDXW)],
                    out_hbm.at[pl.ds(hb * _HR, _HR), :, pl.ds(bb, 1)],
                    o_sems[p])
            return carry

        lax.fori_loop(0, u_per_w // 2, outer, 0)

        for p in range(2):
            pltpu.make_async_copy(
                val_v.at[p, :, :, :, :, pl.ds(0, _IDXW)],
                out_hbm.at[pl.ds(0, _HR), :, pl.ds(0, 1)],
                o_sems[p]).wait()

    return emb


def kernel(feature_ids, table):
    batch, hist = feature_ids.shape
    V1, D = table.shape
    ids32 = feature_ids.astype(jnp.int32)
    i4 = ids32.reshape(batch // _IDXW, _IDXW, hist // _HR, _HR)
    i4 = i4.transpose(2, 0, 3, 1)
    info = plsc.get_sparse_core_info()
    emb = _build(V1, D, batch, hist, info.num_cores, info.num_subcores)
    out5 = emb(i4, table)
    out = out5.transpose(2, 4, 0, 1, 3).reshape(batch, hist, D)
    return out
